# Initial kernel scaffold; baseline (speedup 1.0000x reference)
import jax
import jax.numpy as jnp
from jax.experimental import pallas as pl


def kernel(x):
    raise NotImplementedError("write your pallas kernel here")


# speedup vs baseline: 42.9116x; 42.9116x over previous
import jax
import jax.numpy as jnp
import numpy as np
from jax.experimental import pallas as pl
from jax.experimental.pallas import tpu as pltpu

SR = 16000
WIN = 400
HOP = 160
NFFT = 512
NMELS = 40
NMFCC = 13
B = 64
L = 160000
NF = (L - WIN) // HOP + 1
ROWS = L // HOP
NBINS = 256
PRE = 0.97
NOUT = 3 * NMFCC


def _build_constants():
    n = np.arange(WIN)
    win = 0.54 - 0.46 * np.cos(2.0 * np.pi * n / WIN)
    k = np.arange(NBINS)
    ang = 2.0 * np.pi * np.outer(n, k) / NFFT
    wc = win[:, None] * np.cos(ang)
    ws = -(win[:, None] * np.sin(ang))
    wcs = np.concatenate([wc, ws], axis=1)

    high_mel = 2595.0 * np.log10(1.0 + (SR / 2.0) / 700.0)
    mel_pts = np.linspace(0.0, high_mel, NMELS + 2)
    hz_pts = 700.0 * (10.0 ** (mel_pts / 2595.0) - 1.0)
    bins = np.floor((NFFT + 1) * hz_pts / SR)
    fb = np.zeros((NMELS, NFFT // 2 + 1))
    for m in range(1, NMELS + 1):
        f_lo, f_c, f_hi = int(bins[m - 1]), int(bins[m]), int(bins[m + 1])
        for q in range(f_lo, f_c):
            fb[m - 1, q] = (q - bins[m - 1]) / (f_c - f_lo)
        for q in range(f_c, f_hi):
            fb[m - 1, q] = (bins[m + 1] - q) / (f_hi - f_c)
    fbt = fb[:, :NBINS].T

    nn = np.arange(NMELS)
    kk = np.arange(NMELS)[:, None]
    C = np.cos(np.pi * kk * (2 * nn + 1) / (2.0 * NMELS))
    scale = np.full((NMELS, 1), np.sqrt(2.0 / NMELS))
    scale[0, 0] = np.sqrt(1.0 / NMELS)
    dct = scale * C
    D = np.zeros((NMELS, NMELS))
    for q in range(1, NMELS - 1):
        D[q, q + 1] = 0.5
        D[q, q - 1] = -0.5
    m0 = dct.T
    m1 = m0 @ D.T
    m2 = m1 @ D.T
    wout = np.concatenate([m0[:, :NMFCC], m1[:, :NMFCC], m2[:, :NMFCC]], axis=1)
    return (wcs.astype(np.float32), fbt.astype(np.float32),
            wout.astype(np.float32))


_WCS, _FBT, _WOUT = _build_constants()


def _mfcc_kernel(x_ref, wcs_ref, fbt_ref, wout_ref, o_ref):
    z = x_ref[0]
    c0 = jnp.concatenate(
        [jnp.zeros((1, 1), jnp.float32), z[:-1, HOP - 1:HOP]], axis=0)
    z_prev = jnp.concatenate([c0, z[:, :HOP - 1]], axis=1)
    py = z - PRE * z_prev

    w = wcs_ref[...]
    y = jnp.dot(py[0:NF, :], w[0:HOP, :],
                preferred_element_type=jnp.float32)
    y = y + jnp.dot(py[1:NF + 1, :], w[HOP:2 * HOP, :],
                    preferred_element_type=jnp.float32)
    y = y + jnp.dot(py[2:NF + 2, :WIN - 2 * HOP], w[2 * HOP:WIN, :],
                    preferred_element_type=jnp.float32)

    re = y[:, :NBINS]
    im = y[:, NBINS:]
    mag = jnp.sqrt(re * re + im * im)
    mel = jnp.dot(mag, fbt_ref[...], preferred_element_type=jnp.float32)
    logm = jnp.log(mel + 1e-20)
    o_ref[0] = jnp.dot(logm, wout_ref[...],
                       preferred_element_type=jnp.float32)


@jax.jit
def kernel(x):
    xr = x.reshape(B, ROWS, HOP)
    return pl.pallas_call(
        _mfcc_kernel,
        grid=(B,),
        in_specs=[
            pl.BlockSpec((1, ROWS, HOP), lambda b: (b, 0, 0)),
            pl.BlockSpec((WIN, 2 * NBINS), lambda b: (0, 0)),
            pl.BlockSpec((NBINS, NMELS), lambda b: (0, 0)),
            pl.BlockSpec((NMELS, NOUT), lambda b: (0, 0)),
        ],
        out_specs=pl.BlockSpec((1, NF, NOUT), lambda b: (b, 0, 0)),
        out_shape=jax.ShapeDtypeStruct((B, NF, NOUT), jnp.float32),
        compiler_params=pltpu.CompilerParams(
            dimension_semantics=("parallel",),
        ),
    )(xr, jnp.asarray(_WCS), jnp.asarray(_FBT), jnp.asarray(_WOUT))
